# Initial kernel scaffold; baseline (speedup 1.0000x reference)
import jax
import jax.numpy as jnp
from jax.experimental import pallas as pl


def kernel(x, edge_index, W1, b1, W2, b2):
    raise NotImplementedError("write your pallas kernel here")


# speedup vs baseline: 21.3360x; 21.3360x over previous
import functools

import jax
import jax.numpy as jnp
from jax import lax
from jax.experimental import pallas as pl
from jax.experimental.pallas import tpu as pltpu
from jax.experimental.pallas import tpu_sc as plsc

N = 10000
E = 320000
F_IN = 128
HID = 128
CLS = 40

NC = 2
NS = 16
NW = NC * NS

CHUNK = 80
ROWS_PER_WORKER = E // CHUNK // NW
TILE_ROWS = 624
REM_ROWS = N - TILE_ROWS * NS
REM_R0 = TILE_ROWS * NS

BLK = 1000


def _sc_mesh():
    return plsc.VectorSubcoreMesh(core_axis_name="c", subcore_axis_name="s")


@functools.partial(
    pl.kernel,
    mesh=_sc_mesh(),
    compiler_params=pltpu.CompilerParams(use_tc_tiling_on_sc=False),
    out_type=jax.ShapeDtypeStruct((NC * N, 8), jnp.float32),
    scratch_types=[
        pltpu.VMEM((ROWS_PER_WORKER, CHUNK), jnp.int32),
        pltpu.VMEM((CHUNK, 8), jnp.float32),
        pltpu.VMEM_SHARED((N, 8), jnp.float32),
    ],
)
def _sc_degree(dst_hbm, z8_hbm, ones_hbm, out_hbm, dst_v, ones_v, acc_sh):
    cid = lax.axis_index("c")
    sid = lax.axis_index("s")
    wid = cid * NS + sid
    r0 = sid * TILE_ROWS
    pltpu.sync_copy(z8_hbm.at[pl.ds(r0, TILE_ROWS)],
                    acc_sh.at[pl.ds(r0, TILE_ROWS)])

    @pl.when(sid == NS - 1)
    def _():
        pltpu.sync_copy(z8_hbm.at[pl.ds(REM_R0, REM_ROWS)],
                        acc_sh.at[pl.ds(REM_R0, REM_ROWS)])

    pltpu.sync_copy(ones_hbm, ones_v)
    pltpu.sync_copy(dst_hbm.at[wid], dst_v)
    plsc.subcore_barrier()

    @pl.loop(0, ROWS_PER_WORKER)
    def _(j):
        pltpu.sync_copy(ones_v, acc_sh.at[dst_v.at[j]], add=True)

    plsc.subcore_barrier()
    pltpu.sync_copy(acc_sh.at[pl.ds(r0, TILE_ROWS)],
                    out_hbm.at[pl.ds(cid * N + r0, TILE_ROWS)])

    @pl.when(sid == NS - 1)
    def _():
        pltpu.sync_copy(acc_sh.at[pl.ds(REM_R0, REM_ROWS)],
                        out_hbm.at[pl.ds(cid * N + REM_R0, REM_ROWS)])


def _make_sc_aggregate(d):
    @functools.partial(
        pl.kernel,
        mesh=_sc_mesh(),
        compiler_params=pltpu.CompilerParams(use_tc_tiling_on_sc=False),
        out_type=jax.ShapeDtypeStruct((NC * N, d), jnp.float32),
        scratch_types=[
            pltpu.VMEM((ROWS_PER_WORKER, CHUNK), jnp.int32),
            pltpu.VMEM((ROWS_PER_WORKER, CHUNK), jnp.int32),
            pltpu.VMEM((CHUNK, d), jnp.float32),
            pltpu.VMEM_SHARED((N, d), jnp.float32),
            pltpu.SemaphoreType.DMA,
        ],
    )
    def agg(y_hbm, src_hbm, dst_hbm, out_hbm, src_v, dst_v, rows_v, acc_sh,
            sem):
        cid = lax.axis_index("c")
        sid = lax.axis_index("s")
        wid = cid * NS + sid
        r0 = sid * TILE_ROWS
        pltpu.sync_copy(y_hbm.at[pl.ds(r0, TILE_ROWS)],
                        acc_sh.at[pl.ds(r0, TILE_ROWS)])

        @pl.when(sid == NS - 1)
        def _():
            pltpu.sync_copy(y_hbm.at[pl.ds(REM_R0, REM_ROWS)],
                            acc_sh.at[pl.ds(REM_R0, REM_ROWS)])

        pltpu.sync_copy(src_hbm.at[wid], src_v)
        pltpu.sync_copy(dst_hbm.at[wid], dst_v)
        plsc.subcore_barrier()

        @pl.loop(0, ROWS_PER_WORKER)
        def _(j):
            pltpu.async_copy(y_hbm.at[src_v.at[j]], rows_v, sem).wait()
            pltpu.sync_copy(rows_v, acc_sh.at[dst_v.at[j]], add=True)

        plsc.subcore_barrier()
        pltpu.sync_copy(acc_sh.at[pl.ds(r0, TILE_ROWS)],
                        out_hbm.at[pl.ds(cid * N + r0, TILE_ROWS)])

        @pl.when(sid == NS - 1)
        def _():
            pltpu.sync_copy(acc_sh.at[pl.ds(REM_R0, REM_ROWS)],
                            out_hbm.at[pl.ds(cid * N + REM_R0, REM_ROWS)])

    return agg


_sc_agg_hid = _make_sc_aggregate(HID)
_sc_agg_cls = _make_sc_aggregate(CLS)


def _tc1_body(d0_ref, d1_ref, x_ref, w1_ref, y1_ref, dinv_ref):
    deg = d0_ref[:, :1] + d1_ref[:, :1] + 1.0
    dinv = lax.rsqrt(deg)
    xw = jnp.dot(x_ref[...], w1_ref[...], preferred_element_type=jnp.float32)
    y1_ref[...] = dinv * xw
    dinv_ref[...] = jnp.broadcast_to(dinv, dinv_ref.shape)


_tc1 = pl.pallas_call(
    _tc1_body,
    grid=(N // BLK,),
    in_specs=[
        pl.BlockSpec((BLK, 8), lambda i: (i, 0)),
        pl.BlockSpec((BLK, 8), lambda i: (i, 0)),
        pl.BlockSpec((BLK, F_IN), lambda i: (i, 0)),
        pl.BlockSpec((F_IN, HID), lambda i: (0, 0)),
    ],
    out_specs=[
        pl.BlockSpec((BLK, HID), lambda i: (i, 0)),
        pl.BlockSpec((BLK, 8), lambda i: (i, 0)),
    ],
    out_shape=[
        jax.ShapeDtypeStruct((N, HID), jnp.float32),
        jax.ShapeDtypeStruct((N, 8), jnp.float32),
    ],
)


def _tc2_body(pa_ref, pb_ref, y1_ref, dinv_ref, b1_ref, w2_ref, y2_ref):
    dinv = dinv_ref[:, :1]
    a1 = pa_ref[...] + pb_ref[...] - y1_ref[...]
    h = jnp.maximum(dinv * a1 + b1_ref[...], 0.0)
    y2_ref[...] = dinv * jnp.dot(h, w2_ref[...],
                                 preferred_element_type=jnp.float32)


_tc2 = pl.pallas_call(
    _tc2_body,
    grid=(N // BLK,),
    in_specs=[
        pl.BlockSpec((BLK, HID), lambda i: (i, 0)),
        pl.BlockSpec((BLK, HID), lambda i: (i, 0)),
        pl.BlockSpec((BLK, HID), lambda i: (i, 0)),
        pl.BlockSpec((BLK, 8), lambda i: (i, 0)),
        pl.BlockSpec((1, HID), lambda i: (0, 0)),
        pl.BlockSpec((HID, CLS), lambda i: (0, 0)),
    ],
    out_specs=pl.BlockSpec((BLK, CLS), lambda i: (i, 0)),
    out_shape=jax.ShapeDtypeStruct((N, CLS), jnp.float32),
)


def _tc3_body(pa_ref, pb_ref, y2_ref, dinv_ref, b2_ref, out_ref):
    dinv = dinv_ref[:, :1]
    z = dinv * (pa_ref[...] + pb_ref[...] - y2_ref[...]) + b2_ref[...]
    m = jnp.max(z, axis=1, keepdims=True)
    lse = m + jnp.log(jnp.sum(jnp.exp(z - m), axis=1, keepdims=True))
    out_ref[...] = z - lse


_tc3 = pl.pallas_call(
    _tc3_body,
    grid=(N // BLK,),
    in_specs=[
        pl.BlockSpec((BLK, CLS), lambda i: (i, 0)),
        pl.BlockSpec((BLK, CLS), lambda i: (i, 0)),
        pl.BlockSpec((BLK, CLS), lambda i: (i, 0)),
        pl.BlockSpec((BLK, 8), lambda i: (i, 0)),
        pl.BlockSpec((1, CLS), lambda i: (0, 0)),
    ],
    out_specs=pl.BlockSpec((BLK, CLS), lambda i: (i, 0)),
    out_shape=jax.ShapeDtypeStruct((N, CLS), jnp.float32),
)


def kernel(x, edge_index, W1, b1, W2, b2):
    ei = edge_index.astype(jnp.int32)
    src2d = ei[0].reshape(NW, ROWS_PER_WORKER, CHUNK)
    dst2d = ei[1].reshape(NW, ROWS_PER_WORKER, CHUNK)
    z8 = jnp.zeros((N, 8), jnp.float32)
    o8 = jnp.ones((CHUNK, 8), jnp.float32)

    degp = _sc_degree(dst2d, z8, o8)
    y1, dinv8 = _tc1(degp[:N], degp[N:], x, W1)
    p1 = _sc_agg_hid(y1, src2d, dst2d)
    y2 = _tc2(p1[:N], p1[N:], y1, dinv8,
              b1.reshape(1, HID), W2)
    p2 = _sc_agg_cls(y2, src2d, dst2d)
    return _tc3(p2[:N], p2[N:], y2, dinv8, b2.reshape(1, CLS))
